# baseline (speedup 1.0000x reference)
import functools

import jax
import jax.numpy as jnp
from jax import lax
from jax.experimental import pallas as pl
from jax.experimental.pallas import tpu as pltpu
from jax.experimental.pallas import tpu_sc as plsc

D = 256
C = 4
N = 4096
L = 16
W = 128
VOL = D * D * D
HROWS = C * VOL // W

_GATHER_DNUMS = jax.lax.GatherDimensionNumbers(
    offset_dims=(), collapsed_slice_dims=(0,), start_index_map=(0,))


def _lane_pick(vec, lane_vec):
    return lax.gather(vec, lane_vec[:, None], _GATHER_DNUMS, (1,),
                      mode=jax.lax.GatherScatterMode.PROMISE_IN_BOUNDS)


def _sc_partials(fix3, mov3, pts):
    info = plsc.get_sparse_core_info()
    nw = info.num_cores * info.num_subcores
    ppw = N // nw
    groups = ppw // L
    mesh = plsc.VectorSubcoreMesh(core_axis_name="c", subcore_axis_name="s")

    @functools.partial(
        pl.kernel,
        out_type=jax.ShapeDtypeStruct((nw, L), jnp.float32),
        mesh=mesh,
        scratch_types=[
            pltpu.VMEM((6 * ppw,), jnp.int32),
            pltpu.VMEM((C, ppw), jnp.int32),
            pltpu.VMEM((C, ppw), jnp.int32),
            pltpu.VMEM((ppw, W), jnp.float32),
            pltpu.VMEM((ppw, W), jnp.float32),
            pltpu.VMEM((ppw, W), jnp.float32),
            pltpu.VMEM((ppw, W), jnp.float32),
            pltpu.VMEM((L,), jnp.float32),
            pltpu.SemaphoreType.DMA,
            pltpu.SemaphoreType.DMA,
        ],
    )
    def k(fix3_hbm, mov3_hbm, pts_hbm, out_hbm,
          pts_v, rowf_v, rown_v, fa_v, na_v, fb_v, nb_v, acc_v, sema, semb):
        fix_rows = fix3_hbm.reshape(HROWS, W)
        mov_rows = mov3_hbm.reshape(HROWS, W)
        wid = lax.axis_index("s") * info.num_cores + lax.axis_index("c")
        base = wid * ppw
        for r in range(6):
            pltpu.sync_copy(pts_hbm.at[pl.ds(r * N + base, ppw)],
                            pts_v.at[pl.ds(r * ppw, ppw)])

        for g in range(groups):
            for seg, row_ref in ((0, rowf_v), (3, rown_v)):
                x = pts_v[pl.ds((seg + 0) * ppw + g * L, L)]
                y = pts_v[pl.ds((seg + 1) * ppw + g * L, L)]
                z = pts_v[pl.ds((seg + 2) * ppw + g * L, L)]
                row = lax.shift_right_logical(x * (D * D) + y * D + z, 7)
                for c in range(C):
                    row_ref[c, pl.ds(g * L, L)] = row + c * (VOL // W)

        bufs = ((fa_v, na_v, sema), (fb_v, nb_v, semb))
        lanes = lax.iota(jnp.int32, L)

        def start(c):
            f_v, n_v, sem = bufs[c % 2]
            d1 = pltpu.async_copy(fix_rows.at[rowf_v.at[c]], f_v, sem)
            d2 = pltpu.async_copy(mov_rows.at[rown_v.at[c]], n_v, sem)
            return d1, d2

        def compute(c, acc):
            f_v, n_v, _ = bufs[c % 2]

            def body(g, acc):
                zf_vec = pts_v[pl.ds(2 * ppw + g * L, L)]
                zn_vec = pts_v[pl.ds(5 * ppw + g * L, L)]
                bf_vec = lax.bitwise_and(zf_vec, L - 1)
                bn_vec = lax.bitwise_and(zn_vec, L - 1)
                cf_vec = lax.bitwise_and(lax.shift_right_logical(zf_vec, 4), 7)
                cn_vec = lax.bitwise_and(lax.shift_right_logical(zn_vec, 4), 7)
                for i in range(L):
                    p = g * L + i
                    chf = f_v[p, pl.ds(cf_vec[i] * L, L)]
                    chn = n_v[p, pl.ds(cn_vec[i] * L, L)]
                    fsp = _lane_pick(chf, jnp.full((L,), bf_vec[i], jnp.int32))
                    msp = _lane_pick(chn, jnp.full((L,), bn_vec[i], jnp.int32))
                    d = fsp - msp
                    acc = acc + jnp.where(lanes == i, d * d, 0.0)
                return acc

            return lax.fori_loop(0, groups, body, acc)

        acc = jnp.zeros((L,), jnp.float32)
        descs = start(0)
        for c in range(C):
            nxt = start(c + 1) if c + 1 < C else None
            descs[0].wait()
            descs[1].wait()
            acc = compute(c, acc)
            descs = nxt
        acc_v[...] = acc
        pltpu.sync_copy(acc_v, out_hbm.at[wid])

    return k(fix3, mov3, pts)


def _finalize_kernel(p_ref, o_ref):
    s = jnp.sum(p_ref[...])
    loss = (0.01 * N - s) * (10000.0 / (2.0 * N))
    o_ref[...] = jnp.broadcast_to(loss, (1, 1))


def kernel(fix_image_feature, moving_image_feature, fixed_points,
           positive_points, negative_points):
    del positive_points
    pts = jnp.concatenate(
        [fixed_points.T.reshape(-1), negative_points.T.reshape(-1)])
    v3shape = (C * D * D // 64, 64 * D // W, W)
    partials = _sc_partials(fix_image_feature.reshape(v3shape),
                            moving_image_feature.reshape(v3shape),
                            pts)
    loss = pl.pallas_call(
        _finalize_kernel,
        out_shape=jax.ShapeDtypeStruct((1, 1), jnp.float32),
    )(partials)
    return loss[0, 0]


# speedup vs baseline: 13.4506x; 13.4506x over previous
import functools

import jax
import jax.numpy as jnp
from jax import lax
from jax.experimental import pallas as pl
from jax.experimental.pallas import tpu as pltpu
from jax.experimental.pallas import tpu_sc as plsc

D = 256
C = 4
N = 4096
L = 16
NROWS = C * D * D

_GATHER_DNUMS = jax.lax.GatherDimensionNumbers(
    offset_dims=(), collapsed_slice_dims=(0,), start_index_map=(0,))


def _lane_pick(vec, lane_vec):
    return lax.gather(vec, lane_vec[:, None], _GATHER_DNUMS, (1,),
                      mode=jax.lax.GatherScatterMode.PROMISE_IN_BOUNDS)


def _sc_partials(fix5, mov5, pts):
    info = plsc.get_sparse_core_info()
    nw = info.num_cores * info.num_subcores
    ppw = N // nw
    hp = ppw // 2
    groups = ppw // L
    nsub = 2 * C
    mesh = plsc.VectorSubcoreMesh(core_axis_name="c", subcore_axis_name="s")

    @functools.partial(
        pl.kernel,
        out_type=jax.ShapeDtypeStruct((nw, L), jnp.float32),
        mesh=mesh,
        scratch_types=[
            pltpu.VMEM((6 * ppw,), jnp.int32),
            pltpu.VMEM((C, ppw), jnp.int32),
            pltpu.VMEM((C, ppw), jnp.int32),
            pltpu.VMEM((ppw, D), jnp.float32),
            pltpu.VMEM((ppw, D), jnp.float32),
            pltpu.VMEM((L,), jnp.float32),
            pltpu.SemaphoreType.DMA,
        ],
    )
    def k(fix5_hbm, mov5_hbm, pts_hbm, out_hbm,
          pts_v, rowf_v, rown_v, f_v, n_v, acc_v, sem):
        fix_rows = fix5_hbm.reshape(NROWS, D)
        mov_rows = mov5_hbm.reshape(NROWS, D)
        wid = lax.axis_index("s") * info.num_cores + lax.axis_index("c")
        base = wid * ppw
        for r in range(6):
            pltpu.sync_copy(pts_hbm.at[pl.ds(r * N + base, ppw)],
                            pts_v.at[pl.ds(r * ppw, ppw)])

        for g in range(groups):
            for seg, row_ref in ((0, rowf_v), (3, rown_v)):
                x = pts_v[pl.ds((seg + 0) * ppw + g * L, L)]
                y = pts_v[pl.ds((seg + 1) * ppw + g * L, L)]
                row = x * D + y
                for c in range(C):
                    row_ref[c, pl.ds(g * L, L)] = row + c * (D * D)

        lanes = lax.iota(jnp.int32, L)

        def round_body(c, acc):
            d1 = pltpu.async_copy(fix_rows.at[rowf_v.at[c]], f_v, sem)
            d2 = pltpu.async_copy(mov_rows.at[rown_v.at[c]], n_v, sem)
            d1.wait()
            d2.wait()
            for g in range(groups):
                zf_vec = pts_v[pl.ds(2 * ppw + g * L, L)]
                zn_vec = pts_v[pl.ds(5 * ppw + g * L, L)]
                bf_vec = lax.bitwise_and(zf_vec, L - 1)
                bn_vec = lax.bitwise_and(zn_vec, L - 1)
                for i in range(L):
                    zf = zf_vec[i]
                    zn = zn_vec[i]
                    chf = f_v[g * L + i, pl.ds((zf >> 4) * L, L)]
                    chn = n_v[g * L + i, pl.ds((zn >> 4) * L, L)]
                    fsp = _lane_pick(chf, jnp.full((L,), bf_vec[i], jnp.int32))
                    msp = _lane_pick(chn, jnp.full((L,), bn_vec[i], jnp.int32))
                    dd = (fsp - msp) * (fsp - msp)
                    acc = acc + jnp.where(lanes == i, dd, 0.0)
            return acc

        acc = lax.fori_loop(0, C, round_body, jnp.zeros((L,), jnp.float32))
        acc_v[...] = acc
        pltpu.sync_copy(acc_v, out_hbm.at[wid])

    return k(fix5, mov5, pts)


def _finalize_kernel(p_ref, o_ref):
    s = jnp.sum(p_ref[...])
    loss = (0.01 * N - s) * (10000.0 / (2.0 * N))
    o_ref[...] = jnp.broadcast_to(loss, (1, 1))


def kernel(fix_image_feature, moving_image_feature, fixed_points,
           positive_points, negative_points):
    del positive_points
    pts = jnp.concatenate(
        [fixed_points.T.reshape(-1), negative_points.T.reshape(-1)])
    partials = _sc_partials(fix_image_feature, moving_image_feature, pts)
    loss = pl.pallas_call(
        _finalize_kernel,
        out_shape=jax.ShapeDtypeStruct((1, 1), jnp.float32),
    )(partials)
    return loss[0, 0]
